# Initial kernel scaffold; baseline (speedup 1.0000x reference)
import jax
import jax.numpy as jnp
from jax.experimental import pallas as pl


def kernel(x, edge_index, W):
    raise NotImplementedError("write your pallas kernel here")


# speedup vs baseline: 5.4676x; 5.4676x over previous
import functools

import jax
import jax.numpy as jnp
from jax import lax
from jax.experimental import pallas as pl
from jax.experimental.pallas import tpu as pltpu
from jax.experimental.pallas import tpu_sc as plsc

_N = 10000
_E = 320000
_D = 128

_NC = 2
_NS = 16
_NW = _NC * _NS
_EPW = _E // _NW
_C = 80
_NCHUNK = _EPW // _C
_NP = 10240
_RPT = _NP // _NS
_ZR = 128


def _agg_body(src_hbm, dst_hbm, x_hbm, out_hbm, sidx, didx, rows, zbuf, acc, sem):
    cid = lax.axis_index("c")
    sid = lax.axis_index("s")
    wid = cid * _NS + sid

    zvec = jnp.zeros((16,), jnp.float32)

    def _zrow(r, carry):
        for g in range(_D // 16):
            zbuf[r, pl.ds(g * 16, 16)] = zvec
        return carry

    lax.fori_loop(0, _ZR, _zrow, 0)
    rbase = sid * _RPT
    for i in range(_RPT // _ZR):
        pltpu.sync_copy(zbuf, acc.at[pl.ds(rbase + i * _ZR, _ZR)])
    plsc.subcore_barrier()

    ebase = wid * _EPW

    def _chunk(i, carry):
        off = ebase + i * _C
        pltpu.sync_copy(src_hbm.at[pl.ds(off, _C)], sidx)
        pltpu.sync_copy(dst_hbm.at[pl.ds(off, _C)], didx)
        pltpu.async_copy(x_hbm.at[sidx], rows, sem).wait()
        pltpu.sync_copy(rows, acc.at[didx], add=True)
        return carry

    lax.fori_loop(0, _NCHUNK, _chunk, 0)
    plsc.subcore_barrier()

    pltpu.sync_copy(acc.at[pl.ds(rbase, _RPT)], out_hbm.at[cid, pl.ds(rbase, _RPT)])


@jax.jit
def _aggregate(src, dst, x):
    k = functools.partial(
        pl.kernel,
        mesh=plsc.VectorSubcoreMesh(core_axis_name="c", subcore_axis_name="s"),
        out_type=jax.ShapeDtypeStruct((_NC, _NP, _D), jnp.float32),
        scratch_types=[
            pltpu.VMEM((_C,), jnp.int32),
            pltpu.VMEM((_C,), jnp.int32),
            pltpu.VMEM((_C, _D), jnp.float32),
            pltpu.VMEM((_ZR, _D), jnp.float32),
            pltpu.VMEM_SHARED((_NP, _D), jnp.float32),
            pltpu.SemaphoreType.DMA,
        ],
    )(_agg_body)
    return k(src, dst, x)


def _mm_body(p0_ref, p1_ref, w_ref, o_ref):
    s = p0_ref[...] + p1_ref[...]
    o_ref[...] = jnp.maximum(
        jnp.dot(s, w_ref[...], preferred_element_type=jnp.float32), 0.0
    )


_BLK = 1024


@jax.jit
def _matmul_relu(p0, p1, W):
    return pl.pallas_call(
        _mm_body,
        grid=(_NP // _BLK,),
        in_specs=[
            pl.BlockSpec((_BLK, _D), lambda i: (i, 0)),
            pl.BlockSpec((_BLK, _D), lambda i: (i, 0)),
            pl.BlockSpec((_D, _D), lambda i: (0, 0)),
        ],
        out_specs=pl.BlockSpec((_BLK, _D), lambda i: (i, 0)),
        out_shape=jax.ShapeDtypeStruct((_N, _D), jnp.float32),
    )(p0, p1, W)


def kernel(x, edge_index, W):
    src = edge_index[0].astype(jnp.int32)
    dst = edge_index[1].astype(jnp.int32)
    partials = _aggregate(src, dst, x)
    return _matmul_relu(partials[0], partials[1], W)


# speedup vs baseline: 7.7320x; 1.4142x over previous
import functools

import jax
import jax.numpy as jnp
from jax import lax
from jax.experimental import pallas as pl
from jax.experimental.pallas import tpu as pltpu
from jax.experimental.pallas import tpu_sc as plsc

_N = 10000
_E = 320000
_D = 128

_NC = 2
_NS = 16
_NW = _NC * _NS
_EPW = _E // _NW
_C = 80
_NCHUNK = _EPW // _C
_NP = 10240
_RPT = _NP // _NS
_ZR = 128


def _agg_body(src_hbm, dst_hbm, x_hbm, out_hbm, acc, isem, gsem0, gsem1):
    pl.run_scoped(
        functools.partial(
            _agg_inner, src_hbm, dst_hbm, x_hbm, out_hbm, acc,
            isem, gsem0, gsem1),
        pltpu.VMEM((_C,), jnp.int32),
        pltpu.VMEM((_C,), jnp.int32),
        pltpu.VMEM((_C,), jnp.int32),
        pltpu.VMEM((_C,), jnp.int32),
        pltpu.VMEM((_C, _D), jnp.float32),
        pltpu.VMEM((_C, _D), jnp.float32),
    )


def _agg_inner(src_hbm, dst_hbm, x_hbm, out_hbm, acc, isem, gsem0, gsem1,
               sidx0, sidx1, didx0, didx1, rows0, rows1):
    cid = lax.axis_index("c")
    sid = lax.axis_index("s")
    wid = cid * _NS + sid
    sidx = (sidx0, sidx1)
    didx = (didx0, didx1)
    rows = (rows0, rows1)
    gsem = (gsem0, gsem1)
    ebase = wid * _EPW

    def _idx_load(i, p):
        off = ebase + i * _C
        pltpu.async_copy(src_hbm.at[pl.ds(off, _C)], sidx[p], isem)
        pltpu.async_copy(dst_hbm.at[pl.ds(off, _C)], didx[p], isem)

    def _idx_wait(p):
        pltpu.make_async_copy(src_hbm.at[pl.ds(0, _C)], sidx[p], isem).wait()
        pltpu.make_async_copy(dst_hbm.at[pl.ds(0, _C)], didx[p], isem).wait()

    def _gather(p):
        return pltpu.async_copy(x_hbm.at[sidx[p]], rows[p], gsem[p])

    def _gather_wait(p):
        pltpu.make_async_copy(x_hbm.at[sidx[p]], rows[p], gsem[p]).wait()

    def _scatter(p):
        pltpu.sync_copy(rows[p], acc.at[didx[p]], add=True)

    _idx_load(0, 0)
    zvec = jnp.zeros((16,), jnp.float32)

    def _zrow(r, carry):
        for g in range(_D // 16):
            rows0[r, pl.ds(g * 16, 16)] = zvec
        return carry

    lax.fori_loop(0, _C, _zrow, 0)
    rbase = sid * _RPT
    for i in range(_RPT // _C):
        pltpu.sync_copy(rows0, acc.at[pl.ds(rbase + i * _C, _C)])
    plsc.subcore_barrier()

    _idx_wait(0)
    _gather(0)
    _idx_load(1, 1)

    def _pair_body(j, carry):
        for p in range(2):
            i = 2 * j + p
            _gather_wait(p)
            _scatter(p)
            _idx_wait(1 - p)
            _gather(1 - p)
            nxt = jnp.minimum(i + 2, _NCHUNK - 1)
            _idx_load(nxt, p)
        return carry

    lax.fori_loop(0, (_NCHUNK - 1) // 2, _pair_body, 0)
    _gather_wait(0)
    _scatter(0)
    _idx_wait(1)
    plsc.subcore_barrier()

    pltpu.sync_copy(acc.at[pl.ds(rbase, _RPT)], out_hbm.at[cid, pl.ds(rbase, _RPT)])


@jax.jit
def _aggregate(src, dst, x):
    k = functools.partial(
        pl.kernel,
        mesh=plsc.VectorSubcoreMesh(core_axis_name="c", subcore_axis_name="s"),
        out_type=jax.ShapeDtypeStruct((_NC, _NP, _D), jnp.float32),
        scratch_types=[
            pltpu.VMEM_SHARED((_NP, _D), jnp.float32),
            pltpu.SemaphoreType.DMA,
            pltpu.SemaphoreType.DMA,
            pltpu.SemaphoreType.DMA,
        ],
    )(_agg_body)
    return k(src, dst, x)


def _mm_body(p0_ref, p1_ref, w_ref, o_ref):
    s = p0_ref[...] + p1_ref[...]
    o_ref[...] = jnp.maximum(
        jnp.dot(s, w_ref[...], preferred_element_type=jnp.float32), 0.0
    )


_BLK = 1024


@jax.jit
def _matmul_relu(p0, p1, W):
    return pl.pallas_call(
        _mm_body,
        grid=(_NP // _BLK,),
        in_specs=[
            pl.BlockSpec((_BLK, _D), lambda i: (i, 0)),
            pl.BlockSpec((_BLK, _D), lambda i: (i, 0)),
            pl.BlockSpec((_D, _D), lambda i: (0, 0)),
        ],
        out_specs=pl.BlockSpec((_BLK, _D), lambda i: (i, 0)),
        out_shape=jax.ShapeDtypeStruct((_N, _D), jnp.float32),
    )(p0, p1, W)


def kernel(x, edge_index, W):
    src = edge_index[0].astype(jnp.int32)
    dst = edge_index[1].astype(jnp.int32)
    partials = _aggregate(src, dst, x)
    return _matmul_relu(partials[0], partials[1], W)


# speedup vs baseline: 10.2359x; 1.3238x over previous
import functools

import jax
import jax.numpy as jnp
from jax import lax
from jax.experimental import pallas as pl
from jax.experimental.pallas import tpu as pltpu
from jax.experimental.pallas import tpu_sc as plsc

_N = 10000
_E = 320000
_D = 128

_NC = 2
_NS = 16
_NW = _NC * _NS
_EPW = _E // _NW
_C = 80
_NCHUNK = _EPW // _C
_NP = 10240
_RPT = _NP // _NS
_ZR = 128


def _agg_body(src_hbm, dst_hbm, x_hbm, out_hbm, acc, isem, gsem0, gsem1):
    pl.run_scoped(
        functools.partial(
            _agg_inner, src_hbm, dst_hbm, x_hbm, out_hbm, acc,
            isem, gsem0, gsem1),
        pltpu.VMEM((_C,), jnp.int32),
        pltpu.VMEM((_C,), jnp.int32),
        pltpu.VMEM((_C,), jnp.int32),
        pltpu.VMEM((_C,), jnp.int32),
        pltpu.VMEM((_C, _D), jnp.float32),
        pltpu.VMEM((_C, _D), jnp.float32),
    )


def _agg_inner(src_hbm, dst_hbm, x_hbm, out_hbm, acc, isem, gsem0, gsem1,
               sidx0, sidx1, didx0, didx1, rows0, rows1):
    cid = lax.axis_index("c")
    sid = lax.axis_index("s")
    wid = cid * _NS + sid
    sidx = (sidx0, sidx1)
    didx = (didx0, didx1)
    rows = (rows0, rows1)
    gsem = (gsem0, gsem1)
    ebase = wid * _EPW

    def _idx_load(i, p):
        off = ebase + i * _C
        pltpu.async_copy(src_hbm.at[pl.ds(off, _C)], sidx[p], isem)
        pltpu.async_copy(dst_hbm.at[pl.ds(off, _C)], didx[p], isem)

    def _idx_wait(p):
        pltpu.make_async_copy(src_hbm.at[pl.ds(0, _C)], sidx[p], isem).wait()
        pltpu.make_async_copy(dst_hbm.at[pl.ds(0, _C)], didx[p], isem).wait()

    def _gather(p):
        return pltpu.async_copy(x_hbm.at[sidx[p]], rows[p], gsem[p])

    def _gather_wait(p):
        pltpu.make_async_copy(x_hbm.at[sidx[p]], rows[p], gsem[p]).wait()

    def _scatter(p):
        pltpu.sync_copy(rows[p], acc.at[didx[p]], add=True)

    _idx_load(0, 0)
    zvec = jnp.zeros((16,), jnp.float32)

    def _zrow(r, carry):
        for g in range(_D // 16):
            rows0[r, pl.ds(g * 16, 16)] = zvec
        return carry

    lax.fori_loop(0, _C, _zrow, 0)
    rbase = sid * _RPT
    for i in range(_RPT // _C):
        pltpu.sync_copy(rows0, acc.at[pl.ds(rbase + i * _C, _C)])
    plsc.subcore_barrier()

    _idx_wait(0)
    _gather(0)
    _idx_load(1, 1)

    def _pair_body(j, carry):
        for p in range(2):
            i = 2 * j + p
            _idx_wait(1 - p)
            _gather(1 - p)
            _gather_wait(p)
            _scatter(p)
            nxt = jnp.minimum(i + 2, _NCHUNK - 1)
            _idx_load(nxt, p)
        return carry

    lax.fori_loop(0, (_NCHUNK - 1) // 2, _pair_body, 0)
    _gather_wait(0)
    _scatter(0)
    _idx_wait(1)
    plsc.subcore_barrier()

    pltpu.sync_copy(acc.at[pl.ds(rbase, _RPT)], out_hbm.at[cid, pl.ds(rbase, _RPT)])


@jax.jit
def _aggregate(src, dst, x):
    k = functools.partial(
        pl.kernel,
        mesh=plsc.VectorSubcoreMesh(core_axis_name="c", subcore_axis_name="s"),
        out_type=jax.ShapeDtypeStruct((_NC, _NP, _D), jnp.float32),
        scratch_types=[
            pltpu.VMEM_SHARED((_NP, _D), jnp.float32),
            pltpu.SemaphoreType.DMA,
            pltpu.SemaphoreType.DMA,
            pltpu.SemaphoreType.DMA,
        ],
    )(_agg_body)
    return k(src, dst, x)


def _mm_body(p0_ref, p1_ref, w_ref, o_ref):
    s = p0_ref[...] + p1_ref[...]
    o_ref[...] = jnp.maximum(
        jnp.dot(s, w_ref[...], preferred_element_type=jnp.float32), 0.0
    )


_BLK = 1024


@jax.jit
def _matmul_relu(p0, p1, W):
    return pl.pallas_call(
        _mm_body,
        grid=(_NP // _BLK,),
        in_specs=[
            pl.BlockSpec((_BLK, _D), lambda i: (i, 0)),
            pl.BlockSpec((_BLK, _D), lambda i: (i, 0)),
            pl.BlockSpec((_D, _D), lambda i: (0, 0)),
        ],
        out_specs=pl.BlockSpec((_BLK, _D), lambda i: (i, 0)),
        out_shape=jax.ShapeDtypeStruct((_N, _D), jnp.float32),
    )(p0, p1, W)


def kernel(x, edge_index, W):
    src = edge_index[0].astype(jnp.int32)
    dst = edge_index[1].astype(jnp.int32)
    partials = _aggregate(src, dst, x)
    return _matmul_relu(partials[0], partials[1], W)


# speedup vs baseline: 11.2281x; 1.0969x over previous
import functools

import jax
import jax.numpy as jnp
from jax import lax
from jax.experimental import pallas as pl
from jax.experimental.pallas import tpu as pltpu
from jax.experimental.pallas import tpu_sc as plsc

_N = 10000
_E = 320000
_D = 128

_NC = 2
_NS = 16
_NW = _NC * _NS
_EPW = _E // _NW
_C = 80
_NCHUNK = _EPW // _C
_NP = 10240
_RPT = _NP // _NS
_ZR = 128


def _agg_body(ei_hbm, x_hbm, out_hbm, acc, isem, gsem0, gsem1):
    pl.run_scoped(
        functools.partial(
            _agg_inner, ei_hbm, x_hbm, out_hbm, acc,
            isem, gsem0, gsem1),
        pltpu.VMEM((_C,), jnp.int32),
        pltpu.VMEM((_C,), jnp.int32),
        pltpu.VMEM((_C,), jnp.int32),
        pltpu.VMEM((_C,), jnp.int32),
        pltpu.VMEM((_C, _D), jnp.float32),
        pltpu.VMEM((_C, _D), jnp.float32),
    )


def _agg_inner(ei_hbm, x_hbm, out_hbm, acc, isem, gsem0, gsem1,
               sidx0, sidx1, didx0, didx1, rows0, rows1):
    cid = lax.axis_index("c")
    sid = lax.axis_index("s")
    wid = cid * _NS + sid
    sidx = (sidx0, sidx1)
    didx = (didx0, didx1)
    rows = (rows0, rows1)
    gsem = (gsem0, gsem1)
    ebase = wid * _EPW

    def _idx_load(i, p):
        off = ebase + i * _C
        pltpu.async_copy(ei_hbm.at[pl.ds(off, _C)], sidx[p], isem)
        pltpu.async_copy(ei_hbm.at[pl.ds(_E + off, _C)], didx[p], isem)

    def _idx_wait(p):
        pltpu.make_async_copy(ei_hbm.at[pl.ds(0, _C)], sidx[p], isem).wait()
        pltpu.make_async_copy(ei_hbm.at[pl.ds(0, _C)], didx[p], isem).wait()

    def _gather(p):
        return pltpu.async_copy(x_hbm.at[sidx[p]], rows[p], gsem[p])

    def _gather_wait(p):
        pltpu.make_async_copy(x_hbm.at[sidx[p]], rows[p], gsem[p]).wait()

    def _scatter(p):
        pltpu.sync_copy(rows[p], acc.at[didx[p]], add=True)

    _idx_load(0, 0)
    zvec = jnp.zeros((16,), jnp.float32)

    def _zrow(r, carry):
        for g in range(_D // 16):
            rows0[r, pl.ds(g * 16, 16)] = zvec
        return carry

    lax.fori_loop(0, _C, _zrow, 0)
    rbase = sid * _RPT
    for i in range(_RPT // _C):
        pltpu.sync_copy(rows0, acc.at[pl.ds(rbase + i * _C, _C)])
    plsc.subcore_barrier()

    _idx_wait(0)
    _gather(0)
    _idx_load(1, 1)

    def _pair_body(j, carry):
        for p in range(2):
            i = 2 * j + p
            _idx_wait(1 - p)
            _gather(1 - p)
            _gather_wait(p)
            _scatter(p)
            nxt = jnp.minimum(i + 2, _NCHUNK - 1)
            _idx_load(nxt, p)
        return carry

    lax.fori_loop(0, (_NCHUNK - 1) // 2, _pair_body, 0)
    _gather_wait(0)
    _scatter(0)
    _idx_wait(1)
    plsc.subcore_barrier()

    pltpu.sync_copy(acc.at[pl.ds(rbase, _RPT)], out_hbm.at[cid, pl.ds(rbase, _RPT)])


@jax.jit
def _aggregate(ei, x):
    k = functools.partial(
        pl.kernel,
        mesh=plsc.VectorSubcoreMesh(core_axis_name="c", subcore_axis_name="s"),
        out_type=jax.ShapeDtypeStruct((_NC, _NP, _D), jnp.float32),
        scratch_types=[
            pltpu.VMEM_SHARED((_NP, _D), jnp.float32),
            pltpu.SemaphoreType.DMA,
            pltpu.SemaphoreType.DMA,
            pltpu.SemaphoreType.DMA,
        ],
    )(_agg_body)
    return k(ei, x)


def _mm_body(p_ref, w_ref, o_ref):
    s = p_ref[0] + p_ref[1]
    o_ref[...] = jnp.maximum(
        jnp.dot(s, w_ref[...], preferred_element_type=jnp.float32), 0.0
    )


_BLK = 1024


@jax.jit
def _matmul_relu(partials, W):
    return pl.pallas_call(
        _mm_body,
        grid=(_NP // _BLK,),
        in_specs=[
            pl.BlockSpec((_NC, _BLK, _D), lambda i: (0, i, 0)),
            pl.BlockSpec((_D, _D), lambda i: (0, 0)),
        ],
        out_specs=pl.BlockSpec((_BLK, _D), lambda i: (i, 0)),
        out_shape=jax.ShapeDtypeStruct((_N, _D), jnp.float32),
    )(partials, W)


def kernel(x, edge_index, W):
    ei = edge_index.astype(jnp.int32).reshape(2 * _E)
    partials = _aggregate(ei, x)
    return _matmul_relu(partials, W)


# speedup vs baseline: 11.9359x; 1.0630x over previous
import functools

import jax
import jax.numpy as jnp
from jax import lax
from jax.experimental import pallas as pl
from jax.experimental.pallas import tpu as pltpu
from jax.experimental.pallas import tpu_sc as plsc

_N = 10000
_E = 320000
_D = 128

_NC = 2
_NS = 16
_NW = _NC * _NS
_EPW = _E // _NW
_C = 80
_NCHUNK = _EPW // _C
_NP = 10240
_RPT = _NP // _NS
_ZR = 128


def _agg_body(ei_hbm, x_hbm, out_hbm, acc, *sems):
    pl.run_scoped(
        functools.partial(_agg_inner, ei_hbm, x_hbm, out_hbm, acc, sems),
        pltpu.VMEM((_C,), jnp.int32),
        pltpu.VMEM((_C,), jnp.int32),
        pltpu.VMEM((_C,), jnp.int32),
        pltpu.VMEM((_C,), jnp.int32),
        pltpu.VMEM((_C,), jnp.int32),
        pltpu.VMEM((_C,), jnp.int32),
        pltpu.VMEM((_C, _D), jnp.float32),
        pltpu.VMEM((_C, _D), jnp.float32),
        pltpu.VMEM((_C, _D), jnp.float32),
    )


def _agg_inner(ei_hbm, x_hbm, out_hbm, acc, sems,
               sidx0, sidx1, sidx2, didx0, didx1, didx2, rows0, rows1, rows2):
    cid = lax.axis_index("c")
    sid = lax.axis_index("s")
    wid = cid * _NS + sid
    sidx = (sidx0, sidx1, sidx2)
    didx = (didx0, didx1, didx2)
    rows = (rows0, rows1, rows2)
    isem = sems[0:3]
    gsem = sems[3:6]
    ebase = wid * _EPW

    def _idx_load(i, p):
        off = ebase + i * _C
        pltpu.async_copy(ei_hbm.at[pl.ds(off, _C)], sidx[p], isem[p])
        pltpu.async_copy(ei_hbm.at[pl.ds(_E + off, _C)], didx[p], isem[p])

    def _idx_wait(p):
        pltpu.make_async_copy(ei_hbm.at[pl.ds(0, _C)], sidx[p], isem[p]).wait()
        pltpu.make_async_copy(ei_hbm.at[pl.ds(0, _C)], didx[p], isem[p]).wait()

    def _gather(p):
        pltpu.async_copy(x_hbm.at[sidx[p]], rows[p], gsem[p])

    def _gather_wait(p):
        pltpu.make_async_copy(x_hbm.at[sidx[p]], rows[p], gsem[p]).wait()

    def _scatter(p):
        pltpu.sync_copy(rows[p], acc.at[didx[p]], add=True)

    _idx_load(0, 0)
    _idx_load(1, 1)
    zvec = jnp.zeros((16,), jnp.float32)

    def _zrow(r, carry):
        for g in range(_D // 16):
            rows2[r, pl.ds(g * 16, 16)] = zvec
        return carry

    lax.fori_loop(0, _C, _zrow, 0)
    rbase = sid * _RPT
    _idx_wait(0)
    _gather(0)
    _idx_wait(1)
    _gather(1)
    for i in range(_RPT // _C):
        pltpu.sync_copy(rows2, acc.at[pl.ds(rbase + i * _C, _C)])
    plsc.subcore_barrier()
    _idx_load(2, 2)
    _idx_wait(2)
    _gather(2)

    def _step(i, a):
        _gather_wait(a)
        _scatter(a)
        nxt = jnp.minimum(i + 3, _NCHUNK - 1)
        _idx_load(nxt, a)
        _idx_wait(a)
        _gather(a)

    def _trip_body(j, carry):
        i = 3 * j
        _step(i, 0)
        _step(i + 1, 1)
        _step(i + 2, 2)
        return carry

    lax.fori_loop(0, _NCHUNK // 3 - 1, _trip_body, 0)
    _step(jnp.int32(_NCHUNK - 5), 0)
    _step(jnp.int32(_NCHUNK - 4), 1)
    _step(jnp.int32(_NCHUNK - 3), 2)
    _gather_wait(0)
    _scatter(0)
    _gather_wait(1)
    _scatter(1)
    _gather_wait(2)
    plsc.subcore_barrier()

    pltpu.sync_copy(acc.at[pl.ds(rbase, _RPT)], out_hbm.at[cid, pl.ds(rbase, _RPT)])


@jax.jit
def _aggregate(ei, x):
    k = functools.partial(
        pl.kernel,
        mesh=plsc.VectorSubcoreMesh(core_axis_name="c", subcore_axis_name="s"),
        out_type=jax.ShapeDtypeStruct((_NC, _NP, _D), jnp.float32),
        scratch_types=[
            pltpu.VMEM_SHARED((_NP, _D), jnp.float32),
        ] + [pltpu.SemaphoreType.DMA] * 6,
    )(_agg_body)
    return k(ei, x)


def _mm_body(p_ref, w_ref, o_ref):
    s = p_ref[0] + p_ref[1]
    o_ref[...] = jnp.maximum(
        jnp.dot(s, w_ref[...], preferred_element_type=jnp.float32), 0.0
    )


_BLK = 1024


@jax.jit
def _matmul_relu(partials, W):
    return pl.pallas_call(
        _mm_body,
        grid=(_NP // _BLK,),
        in_specs=[
            pl.BlockSpec((_NC, _BLK, _D), lambda i: (0, i, 0)),
            pl.BlockSpec((_D, _D), lambda i: (0, 0)),
        ],
        out_specs=pl.BlockSpec((_BLK, _D), lambda i: (i, 0)),
        out_shape=jax.ShapeDtypeStruct((_N, _D), jnp.float32),
    )(partials, W)


def kernel(x, edge_index, W):
    ei = edge_index.astype(jnp.int32).reshape(2 * _E)
    partials = _aggregate(ei, x)
    return _matmul_relu(partials, W)


# speedup vs baseline: 12.9744x; 1.0870x over previous
import functools

import jax
import jax.numpy as jnp
from jax import lax
from jax.experimental import pallas as pl
from jax.experimental.pallas import tpu as pltpu
from jax.experimental.pallas import tpu_sc as plsc

_N = 10000
_E = 320000
_D = 128

_NC = 2
_NS = 16
_NW = _NC * _NS
_EPW = _E // _NW
_C = 80
_NCHUNK = _EPW // _C
_NP = 10240
_RPT = _NP // _NS
_ZR = 128


def _agg_body(ei_hbm, x_hbm, out_hbm, acc, *sems):
    pl.run_scoped(
        functools.partial(_agg_inner, ei_hbm, x_hbm, out_hbm, acc, sems),
        pltpu.VMEM((_C,), jnp.int32),
        pltpu.VMEM((_C,), jnp.int32),
        pltpu.VMEM((_C,), jnp.int32),
        pltpu.VMEM((_C,), jnp.int32),
        pltpu.VMEM((_C,), jnp.int32),
        pltpu.VMEM((_C,), jnp.int32),
        pltpu.VMEM((_C, _D), jnp.float32),
        pltpu.VMEM((_C, _D), jnp.float32),
        pltpu.VMEM((_C, _D), jnp.float32),
    )


def _agg_inner(ei_hbm, x_hbm, out_hbm, acc, sems,
               sidx0, sidx1, sidx2, didx0, didx1, didx2, rows0, rows1, rows2):
    cid = lax.axis_index("c")
    sid = lax.axis_index("s")
    wid = cid * _NS + sid
    sidx = (sidx0, sidx1, sidx2)
    didx = (didx0, didx1, didx2)
    rows = (rows0, rows1, rows2)
    isem = sems[0:3]
    gsem = sems[3:6]
    ssem = sems[6:9]
    ebase = wid * _EPW

    def _idx_load(i, p):
        off = ebase + i * _C
        pltpu.async_copy(ei_hbm.at[pl.ds(off, _C)], sidx[p], isem[p])
        pltpu.async_copy(ei_hbm.at[pl.ds(_E + off, _C)], didx[p], isem[p])

    def _idx_wait(p):
        pltpu.make_async_copy(ei_hbm.at[pl.ds(0, _C)], sidx[p], isem[p]).wait()
        pltpu.make_async_copy(ei_hbm.at[pl.ds(0, _C)], didx[p], isem[p]).wait()

    def _gather(p):
        pltpu.async_copy(x_hbm.at[sidx[p]], rows[p], gsem[p])

    def _gather_wait(p):
        pltpu.make_async_copy(x_hbm.at[sidx[p]], rows[p], gsem[p]).wait()

    def _scatter(p):
        pltpu.async_copy(rows[p], acc.at[didx[p]], ssem[p], add=True)

    def _scatter_wait(p):
        pltpu.make_async_copy(rows[p], acc.at[didx[p]], ssem[p]).wait()

    _idx_load(0, 0)
    _idx_load(1, 1)
    zvec = jnp.zeros((16,), jnp.float32)

    def _zrow(r, carry):
        for g in range(_D // 16):
            rows2[r, pl.ds(g * 16, 16)] = zvec
        return carry

    lax.fori_loop(0, _C, _zrow, 0)
    rbase = sid * _RPT
    _idx_wait(0)
    _gather(0)
    _idx_wait(1)
    _gather(1)
    for i in range(_RPT // _C):
        pltpu.sync_copy(rows2, acc.at[pl.ds(rbase + i * _C, _C)])
    plsc.subcore_barrier()
    _idx_load(2, 2)

    def _step(i, a, first=False):
        b = (a + 2) % 3
        _gather_wait(a)
        if not first:
            _scatter_wait(b)
        _scatter(a)
        nxt = jnp.minimum(i + 2, _NCHUNK - 1)
        _idx_load(nxt, b)
        _idx_wait(b)
        _gather(b)

    _gather_wait(0)
    _scatter(0)
    _idx_wait(2)
    _gather(2)

    def _trip_body(j, carry):
        i = 3 * j
        _step(i + 1, 1)
        _step(i + 2, 2)
        _step(i + 3, 0)
        return carry

    lax.fori_loop(0, (_NCHUNK - 2) // 3, _trip_body, 0)
    _gather_wait(1)
    _scatter_wait(0)
    _scatter(1)
    _scatter_wait(1)
    _gather_wait(2)
    plsc.subcore_barrier()

    pltpu.sync_copy(acc.at[pl.ds(rbase, _RPT)], out_hbm.at[cid, pl.ds(rbase, _RPT)])


@jax.jit
def _aggregate(ei, x):
    k = functools.partial(
        pl.kernel,
        mesh=plsc.VectorSubcoreMesh(core_axis_name="c", subcore_axis_name="s"),
        out_type=jax.ShapeDtypeStruct((_NC, _NP, _D), jnp.float32),
        scratch_types=[
            pltpu.VMEM_SHARED((_NP, _D), jnp.float32),
        ] + [pltpu.SemaphoreType.DMA] * 9,
    )(_agg_body)
    return k(ei, x)


def _mm_body(p_ref, w_ref, o_ref):
    s = p_ref[0] + p_ref[1]
    o_ref[...] = jnp.maximum(
        jnp.dot(s, w_ref[...], preferred_element_type=jnp.float32), 0.0
    )


_BLK = 1024


@jax.jit
def _matmul_relu(partials, W):
    return pl.pallas_call(
        _mm_body,
        grid=(_NP // _BLK,),
        in_specs=[
            pl.BlockSpec((_NC, _BLK, _D), lambda i: (0, i, 0)),
            pl.BlockSpec((_D, _D), lambda i: (0, 0)),
        ],
        out_specs=pl.BlockSpec((_BLK, _D), lambda i: (i, 0)),
        out_shape=jax.ShapeDtypeStruct((_N, _D), jnp.float32),
    )(partials, W)


def kernel(x, edge_index, W):
    ei = edge_index.astype(jnp.int32).reshape(2 * _E)
    partials = _aggregate(ei, x)
    return _matmul_relu(partials, W)


# speedup vs baseline: 15.3042x; 1.1796x over previous
import functools

import jax
import jax.numpy as jnp
from jax import lax
from jax.experimental import pallas as pl
from jax.experimental.pallas import tpu as pltpu
from jax.experimental.pallas import tpu_sc as plsc

_N = 10000
_E = 320000
_D = 128

_NC = 2
_NS = 16
_NW = _NC * _NS
_EPW = _E // _NW
_C = 80
_NCHUNK = _EPW // _C
_NP = 10240
_RPT = _NP // _NS
_ZR = 128


def _agg_body(ei_hbm, x_hbm, out_hbm, acc, *sems):
    pl.run_scoped(
        functools.partial(_agg_inner, ei_hbm, x_hbm, out_hbm, acc, sems),
        *([pltpu.VMEM((_C,), jnp.int32)] * 12),
        pltpu.VMEM((_C, _D), jnp.float32),
        pltpu.VMEM((_C, _D), jnp.float32),
        pltpu.VMEM((_C, _D), jnp.float32),
    )


def _agg_inner(ei_hbm, x_hbm, out_hbm, acc, sems, *bufs):
    cid = lax.axis_index("c")
    sid = lax.axis_index("s")
    wid = cid * _NS + sid
    sidx = bufs[0:6]
    didx = bufs[6:12]
    rows = bufs[12:15]
    isem = sems[0:6]
    gsem = sems[6:9]
    ssem = sems[9:12]
    ebase = wid * _EPW

    def _idx_load(i, q):
        off = ebase + i * _C
        pltpu.async_copy(ei_hbm.at[pl.ds(off, _C)], sidx[q], isem[q])
        pltpu.async_copy(ei_hbm.at[pl.ds(_E + off, _C)], didx[q], isem[q])

    def _idx_wait(q):
        pltpu.make_async_copy(ei_hbm.at[pl.ds(0, _C)], sidx[q], isem[q]).wait()
        pltpu.make_async_copy(ei_hbm.at[pl.ds(0, _C)], didx[q], isem[q]).wait()

    def _gather(a, q):
        pltpu.async_copy(x_hbm.at[sidx[q]], rows[a], gsem[a])

    def _gather_wait(a, q):
        pltpu.make_async_copy(x_hbm.at[sidx[q]], rows[a], gsem[a]).wait()

    def _scatter(a, q):
        pltpu.async_copy(rows[a], acc.at[didx[q]], ssem[a], add=True)

    def _scatter_wait(a, q):
        pltpu.make_async_copy(rows[a], acc.at[didx[q]], ssem[a]).wait()

    _idx_load(0, 0)
    _idx_load(1, 1)
    _idx_load(2, 2)
    zvec = jnp.zeros((16,), jnp.float32)

    def _zrow(r, carry):
        for g in range(_D // 16):
            rows[2][r, pl.ds(g * 16, 16)] = zvec
        return carry

    lax.fori_loop(0, _C, _zrow, 0)
    rbase = sid * _RPT
    _idx_wait(0)
    _gather(0, 0)
    _idx_wait(1)
    _gather(1, 1)
    for i in range(_RPT // _C):
        pltpu.sync_copy(rows[2], acc.at[pl.ds(rbase + i * _C, _C)])
    plsc.subcore_barrier()

    def _step(i, k, first=False):
        a, q = k % 3, k % 6
        _idx_load(jnp.minimum(i + 3, _NCHUNK - 1), (k + 3) % 6)
        _gather_wait(a, q)
        if not first:
            _scatter_wait((k + 2) % 3, (k + 5) % 6)
        _scatter(a, q)
        _idx_wait((k + 2) % 6)
        _gather((k + 2) % 3, (k + 2) % 6)

    _step(jnp.int32(0), 0, first=True)
    _step(jnp.int32(1), 1)
    _step(jnp.int32(2), 2)
    _step(jnp.int32(3), 3)
    _step(jnp.int32(4), 4)

    def _hex_body(j, carry):
        i = 6 * j + 5
        for k in range(6):
            _step(i + k, 5 + k)
        return carry

    lax.fori_loop(0, (_NCHUNK - 5) // 6, _hex_body, 0)
    _scatter_wait(1, 4)
    _gather_wait(2, 5)
    _gather_wait(0, 0)
    _idx_wait(1)
    plsc.subcore_barrier()

    pltpu.sync_copy(acc.at[pl.ds(rbase, _RPT)], out_hbm.at[cid, pl.ds(rbase, _RPT)])


@jax.jit
def _aggregate(ei, x):
    k = functools.partial(
        pl.kernel,
        mesh=plsc.VectorSubcoreMesh(core_axis_name="c", subcore_axis_name="s"),
        out_type=jax.ShapeDtypeStruct((_NC, _NP, _D), jnp.float32),
        scratch_types=[
            pltpu.VMEM_SHARED((_NP, _D), jnp.float32),
        ] + [pltpu.SemaphoreType.DMA] * 12,
    )(_agg_body)
    return k(ei, x)


def _mm_body(p_ref, w_ref, o_ref):
    s = p_ref[0] + p_ref[1]
    o_ref[...] = jnp.maximum(
        jnp.dot(s, w_ref[...], preferred_element_type=jnp.float32), 0.0
    )


_BLK = 1024


@jax.jit
def _matmul_relu(partials, W):
    return pl.pallas_call(
        _mm_body,
        grid=(_NP // _BLK,),
        in_specs=[
            pl.BlockSpec((_NC, _BLK, _D), lambda i: (0, i, 0)),
            pl.BlockSpec((_D, _D), lambda i: (0, 0)),
        ],
        out_specs=pl.BlockSpec((_BLK, _D), lambda i: (i, 0)),
        out_shape=jax.ShapeDtypeStruct((_N, _D), jnp.float32),
    )(partials, W)


def kernel(x, edge_index, W):
    ei = edge_index.astype(jnp.int32).reshape(2 * _E)
    partials = _aggregate(ei, x)
    return _matmul_relu(partials, W)
